# speedup vs baseline: 1.4681x; 1.4681x over previous
import functools

import jax
import jax.numpy as jnp
from jax import lax
from jax.experimental import pallas as pl
from jax.experimental.pallas import tpu as pltpu
from jax.experimental.pallas import tpu_sc as plsc

_BATCH = 16384
_FIELDS = 100
_N = _BATCH * _FIELDS
_LANES = 16
_NUM_WORKERS = 32
_PER_W = _N // _NUM_WORKERS
_UNROLL = 8
_STEPS = _PER_W // (_LANES * _UNROLL)

_mesh = plsc.VectorSubcoreMesh(core_axis_name="c", subcore_axis_name="s")


@functools.partial(
    pl.kernel,
    mesh=_mesh,
    out_type=jax.ShapeDtypeStruct((_NUM_WORKERS, _LANES), jnp.float32),
    scratch_types=[
        pltpu.VMEM((_PER_W,), jnp.int32),
        pltpu.VMEM((_PER_W,), jnp.float32),
        pltpu.VMEM((_LANES,), jnp.float32),
        pltpu.SemaphoreType.DMA,
    ],
)
def _gather_sum(idx_hbm, w_hbm, out_hbm, idx_v, vals_v, acc_v, sem):
    nc = plsc.get_sparse_core_info().num_cores
    wid = lax.axis_index("s") * nc + lax.axis_index("c")
    base = wid * _PER_W
    pltpu.sync_copy(idx_hbm.at[pl.ds(base, _PER_W)], idx_v)
    pltpu.async_copy(w_hbm.at[idx_v], vals_v, sem).wait()

    def body(i, accs):
        o = i * (_LANES * _UNROLL)
        return tuple(
            accs[j] + vals_v[pl.ds(o + j * _LANES, _LANES)]
            for j in range(_UNROLL)
        )

    zeros = jnp.zeros((_LANES,), jnp.float32)
    accs = lax.fori_loop(0, _STEPS, body, (zeros,) * _UNROLL)
    total = accs[0]
    for j in range(1, _UNROLL):
        total = total + accs[j]
    acc_v[...] = total
    pltpu.sync_copy(acc_v, out_hbm.at[wid])


def kernel(non_zero_indices, weights):
    idx = non_zero_indices.reshape(-1).astype(jnp.int32)
    partials = _gather_sum(idx, weights)
    return jnp.sum(partials)
